# baseline (speedup 1.0000x reference)
import functools
import math

import jax
import jax.numpy as jnp
import numpy as np
from jax import lax
from jax.experimental import pallas as pl
from jax.experimental.pallas import tpu as pltpu
from jax.experimental.pallas import tpu_sc as plsc

_K = 2
_CAP_FACTOR = 1.25


def _router_body(C, x_ref, wg_ref, eidx_ref, gates_ref, mask_ref, slot_ref,
                 w_ref):
    x = x_ref[...]
    wg = wg_ref[...]
    T, E = x.shape[0], wg.shape[1]
    logits = jnp.dot(x, wg, preferred_element_type=jnp.float32)
    lane = lax.broadcasted_iota(jnp.int32, logits.shape, 1)
    m1 = jnp.max(logits, axis=1, keepdims=True)
    i1 = jnp.min(jnp.where(logits == m1, lane, E), axis=1, keepdims=True)
    logits2 = jnp.where(lane == i1, jnp.float32(-jnp.inf), logits)
    m2 = jnp.max(logits2, axis=1, keepdims=True)
    i2 = jnp.min(jnp.where(logits2 == m2, lane, E), axis=1, keepdims=True)
    e2 = jnp.exp(m2 - m1)
    den = 1.0 + e2
    g1 = 1.0 / den
    g2 = e2 / den
    oh1 = (lane == i1).astype(jnp.float32)
    oh2 = (lane == i2).astype(jnp.float32)
    per = oh1 + oh2
    B = 512
    r = lax.broadcasted_iota(jnp.int32, (B, B), 0)
    c = lax.broadcasted_iota(jnp.int32, (B, B), 1)
    ltri = (r > c).astype(jnp.float32)
    blocks = []
    carry = jnp.zeros((1, E), jnp.float32)
    for b in range(T // B):
        pb = per[b * B:(b + 1) * B]
        blocks.append(jnp.dot(ltri, pb, preferred_element_type=jnp.float32)
                      + carry)
        carry = carry + jnp.sum(pb, axis=0, keepdims=True)
    s = jnp.concatenate(blocks, axis=0)
    pos1 = jnp.sum(s * oh1, axis=1, keepdims=True)
    pos2 = jnp.sum(s * oh2, axis=1, keepdims=True)
    k1 = pos1 < float(C)
    k2 = pos2 < float(C)
    p1 = jnp.minimum(pos1, float(C - 1)).astype(jnp.int32)
    p2 = jnp.minimum(pos2, float(C - 1)).astype(jnp.int32)
    n_rows = E * C
    cs1 = jnp.where(k1, i1 * C + p1, n_rows)
    cs2 = jnp.where(k2, i2 * C + p2, n_rows)
    eidx_ref[...] = jnp.concatenate([i1, i2], axis=1)
    gates_ref[...] = jnp.concatenate([g1, g2], axis=1)
    mask_ref[...] = jnp.concatenate(
        [k1.astype(jnp.int32), k2.astype(jnp.int32)], axis=1)
    slot_ref[...] = jnp.concatenate([cs1, cs2], axis=1)
    w_ref[...] = jnp.concatenate(
        [g1 * k1.astype(jnp.float32), g2 * k2.astype(jnp.float32)], axis=1)


def _router(x, Wg, C):
    T = x.shape[0]
    outs = pl.pallas_call(
        functools.partial(_router_body, C),
        out_shape=[
            jax.ShapeDtypeStruct((T, _K), jnp.int32),
            jax.ShapeDtypeStruct((T, _K), jnp.float32),
            jax.ShapeDtypeStruct((T, _K), jnp.int32),
            jax.ShapeDtypeStruct((T, _K), jnp.int32),
            jax.ShapeDtypeStruct((T, _K), jnp.float32),
        ],
    )(x, Wg)
    return outs


def _dispatch(x, cslot_f, w_f, n_rows):
    T, D = x.shape
    n_pair = cslot_f.shape[0]
    info = plsc.get_sparse_core_info()
    nw = info.num_cores * info.num_subcores
    tok_per_w = T // nw
    pairs_per_w = n_pair // nw
    mesh = plsc.VectorSubcoreMesh(core_axis_name="c", subcore_axis_name="s")

    @functools.partial(
        pl.kernel,
        mesh=mesh,
        out_type=[
            jax.ShapeDtypeStruct((n_rows + 8, D), jnp.float32),
            jax.ShapeDtypeStruct((n_rows + 8,), jnp.float32),
        ],
        scratch_types=[
            pltpu.VMEM((pairs_per_w,), jnp.int32),
            pltpu.VMEM((pairs_per_w,), jnp.float32),
            pltpu.VMEM((tok_per_w,), jnp.int32),
            pltpu.VMEM((tok_per_w,), jnp.int32),
            pltpu.VMEM((tok_per_w,), jnp.float32),
            pltpu.VMEM((tok_per_w,), jnp.float32),
            pltpu.VMEM((tok_per_w, D), jnp.float32),
            pltpu.SemaphoreType.DMA,
        ],
        compiler_params=pltpu.CompilerParams(needs_layout_passes=False),
    )
    def k(x_hbm, slot_hbm, w_hbm, xe_hbm, wslot_hbm, slot_v, w_v, idx0_v,
          idx1_v, w0_v, w1_v, rows_v, sem):
        wid = lax.axis_index("s") * info.num_cores + lax.axis_index("c")
        pb = pl.multiple_of(wid * pairs_per_w, pairs_per_w)
        tb = pl.multiple_of(wid * tok_per_w, tok_per_w)
        cp = pltpu.async_copy(x_hbm.at[pl.ds(tb, tok_per_w)], rows_v, sem)
        pltpu.sync_copy(slot_hbm.at[pl.ds(pb, pairs_per_w)], slot_v)
        pltpu.sync_copy(w_hbm.at[pl.ds(pb, pairs_per_w)], w_v)
        lanes = lax.broadcasted_iota(jnp.int32, (16,), 0)
        for j in range(tok_per_w // 16):
            two = jnp.full((16,), 2, jnp.int32)
            src_idx0 = jnp.full((16,), 32 * j, jnp.int32) + lanes * two
            src_idx1 = src_idx0 + 1
            idx0_v[pl.ds(j * 16, 16)] = plsc.load_gather(slot_v, [src_idx0])
            idx1_v[pl.ds(j * 16, 16)] = plsc.load_gather(slot_v, [src_idx1])
            w0_v[pl.ds(j * 16, 16)] = plsc.load_gather(w_v, [src_idx0])
            w1_v[pl.ds(j * 16, 16)] = plsc.load_gather(w_v, [src_idx1])
        pltpu.sync_copy(w0_v, wslot_hbm.at[idx0_v])
        pltpu.sync_copy(w1_v, wslot_hbm.at[idx1_v])
        cp.wait()
        pltpu.sync_copy(rows_v, xe_hbm.at[idx0_v])
        pltpu.sync_copy(rows_v, xe_hbm.at[idx1_v])

    return k(x, cslot_f, w_f)


def _ffn_body(xe_ref, w1_ref, b1_ref, w2_ref, b2_ref, ws_ref, out_ref):
    xb = xe_ref[...]
    h = jnp.dot(xb, w1_ref[0], preferred_element_type=jnp.float32)
    h = jnp.maximum(h + b1_ref[0], 0.0)
    y = jnp.dot(h, w2_ref[0], preferred_element_type=jnp.float32) + b2_ref[0]
    out_ref[...] = y * ws_ref[...]


def _ffn(xe, W1, b1, W2, b2, w_slot, C):
    E, D, F = W1.shape
    out = pl.pallas_call(
        _ffn_body,
        grid=(E,),
        in_specs=[
            pl.BlockSpec((C, D), lambda e: (e, 0)),
            pl.BlockSpec((1, D, F), lambda e: (e, 0, 0)),
            pl.BlockSpec((1, 1, F), lambda e: (e, 0, 0)),
            pl.BlockSpec((1, F, D), lambda e: (e, 0, 0)),
            pl.BlockSpec((1, 1, D), lambda e: (e, 0, 0)),
            pl.BlockSpec((C, 1), lambda e: (e, 0)),
        ],
        out_specs=pl.BlockSpec((C, D), lambda e: (e, 0)),
        out_shape=jax.ShapeDtypeStruct((E * C + 8, D), jnp.float32),
        compiler_params=pltpu.CompilerParams(
            dimension_semantics=("arbitrary",)),
    )(xe, W1, b1.reshape(E, 1, F), W2, b2.reshape(E, 1, D),
      w_slot.reshape(-1, 1))
    return out


def _combine(yw, cslot_f, T):
    n_rows_pad, D = yw.shape
    n_rows = n_rows_pad - 8
    info = plsc.get_sparse_core_info()
    nw = info.num_cores * info.num_subcores
    tok_per_w = T // nw
    pairs_per_w = _K * tok_per_w
    tch = 32
    n_ch = tok_per_w // tch
    mesh = plsc.VectorSubcoreMesh(core_axis_name="c", subcore_axis_name="s")

    @functools.partial(
        pl.kernel,
        mesh=mesh,
        out_type=jax.ShapeDtypeStruct((T, D), jnp.float32),
        scratch_types=[
            pltpu.VMEM((pairs_per_w,), jnp.int32),
            pltpu.VMEM((tok_per_w,), jnp.int32),
            pltpu.VMEM((tok_per_w,), jnp.int32),
            pltpu.VMEM((tch, D), jnp.float32),
            pltpu.VMEM((tch, D), jnp.float32),
            pltpu.VMEM((8, D), jnp.float32),
            pltpu.SemaphoreType.DMA,
            pltpu.SemaphoreType.DMA,
        ],
        compiler_params=pltpu.CompilerParams(needs_layout_passes=False),
    )
    def k(yw_hbm, slot_hbm, out_hbm, slot_v, idx0_v, idx1_v, r0_v, r1_v,
          z_v, sem0, sem1):
        sid = lax.axis_index("s")
        wid = sid * info.num_cores + lax.axis_index("c")
        pb = pl.multiple_of(wid * pairs_per_w, pairs_per_w)
        pltpu.sync_copy(slot_hbm.at[pl.ds(pb, pairs_per_w)], slot_v)
        @pl.when(sid == 0)
        def _():
            zz = jnp.zeros((16,), jnp.float32)

            def zb(i, carry):
                o = pl.multiple_of(i * 16, 16)
                for r in range(8):
                    z_v[r, pl.ds(o, 16)] = zz
                return carry

            lax.fori_loop(0, D // 16, zb, 0)
            pltpu.sync_copy(z_v, yw_hbm.at[pl.ds(n_rows, 8)])

        lanes = lax.broadcasted_iota(jnp.int32, (16,), 0)
        for j in range(tok_per_w // 16):
            two = jnp.full((16,), 2, jnp.int32)
            src_idx0 = jnp.full((16,), 32 * j, jnp.int32) + lanes * two
            src_idx1 = src_idx0 + 1
            idx0_v[pl.ds(j * 16, 16)] = plsc.load_gather(slot_v, [src_idx0])
            idx1_v[pl.ds(j * 16, 16)] = plsc.load_gather(slot_v, [src_idx1])
        plsc.subcore_barrier()
        for ch in range(n_ch):
            co = pl.multiple_of(ch * tch, tch)
            cp0 = pltpu.async_copy(
                yw_hbm.at[idx0_v.at[pl.ds(co, tch)]], r0_v, sem0)
            cp1 = pltpu.async_copy(
                yw_hbm.at[idx1_v.at[pl.ds(co, tch)]], r1_v, sem1)
            cp0.wait()
            cp1.wait()

            def ab(j, carry):
                o = pl.multiple_of(j * 16, 16)
                for t in range(tch):
                    r0_v[t, pl.ds(o, 16)] = (r0_v[t, pl.ds(o, 16)]
                                             + r1_v[t, pl.ds(o, 16)])
                return carry

            lax.fori_loop(0, D // 16, ab, 0)
            tb = pl.multiple_of(wid * tok_per_w + ch * tch, tch)
            pltpu.sync_copy(r0_v, out_hbm.at[pl.ds(tb, tch)])

    return k(yw, cslot_f)


def kernel(x, Wg, W1, b1, W2, b2):
    T, D = x.shape
    E = Wg.shape[1]
    C = int(math.ceil(T * _K / E * _CAP_FACTOR))
    eidx, gates, maski, cslot, wcomb = _router(x, Wg, C)
    cslot_f = cslot.reshape(-1)
    xe, w_slot = _dispatch(x, cslot_f, wcomb.reshape(-1), E * C)
    yw = _ffn(xe, W1, b1, W2, b2, w_slot, C)
    out = _combine(yw, cslot_f, T)
    return out, maski.astype(bool), eidx, gates


# speedup vs baseline: 1.2982x; 1.2982x over previous
import functools
import math

import jax
import jax.numpy as jnp
import numpy as np
from jax import lax
from jax.experimental import pallas as pl
from jax.experimental.pallas import tpu as pltpu
from jax.experimental.pallas import tpu_sc as plsc

_K = 2
_CAP_FACTOR = 1.25


def _router_body(C, x_ref, wg_ref, eidx_ref, gates_ref, mask_ref, slot_ref,
                 w_ref):
    x = x_ref[...]
    wg = wg_ref[...]
    T, E = x.shape[0], wg.shape[1]
    logits = jnp.dot(x, wg, preferred_element_type=jnp.float32)
    lane = lax.broadcasted_iota(jnp.int32, logits.shape, 1)
    m1 = jnp.max(logits, axis=1, keepdims=True)
    i1 = jnp.min(jnp.where(logits == m1, lane, E), axis=1, keepdims=True)
    logits2 = jnp.where(lane == i1, jnp.float32(-jnp.inf), logits)
    m2 = jnp.max(logits2, axis=1, keepdims=True)
    i2 = jnp.min(jnp.where(logits2 == m2, lane, E), axis=1, keepdims=True)
    e2 = jnp.exp(m2 - m1)
    den = 1.0 + e2
    g1 = 1.0 / den
    g2 = e2 / den
    oh1 = (lane == i1).astype(jnp.float32)
    oh2 = (lane == i2).astype(jnp.float32)
    per = oh1 + oh2
    B = 512
    r = lax.broadcasted_iota(jnp.int32, (B, B), 0)
    c = lax.broadcasted_iota(jnp.int32, (B, B), 1)
    ltri = (r > c).astype(jnp.float32)
    blocks = []
    carry = jnp.zeros((1, E), jnp.float32)
    for b in range(T // B):
        pb = per[b * B:(b + 1) * B]
        blocks.append(jnp.dot(ltri, pb, preferred_element_type=jnp.float32)
                      + carry)
        carry = carry + jnp.sum(pb, axis=0, keepdims=True)
    s = jnp.concatenate(blocks, axis=0)
    pos1 = jnp.sum(s * oh1, axis=1, keepdims=True)
    pos2 = jnp.sum(s * oh2, axis=1, keepdims=True)
    k1 = pos1 < float(C)
    k2 = pos2 < float(C)
    p1 = jnp.minimum(pos1, float(C - 1)).astype(jnp.int32)
    p2 = jnp.minimum(pos2, float(C - 1)).astype(jnp.int32)
    eidx_ref[...] = jnp.concatenate([i1, i2], axis=1)
    gates_ref[...] = jnp.concatenate([g1, g2], axis=1)
    mask_ref[...] = jnp.concatenate(
        [k1.astype(jnp.int32), k2.astype(jnp.int32)], axis=1)
    slot_ref[...] = jnp.concatenate([i1 * C + p1, i2 * C + p2], axis=1)
    w_ref[...] = jnp.concatenate(
        [g1 * k1.astype(jnp.float32), g2 * k2.astype(jnp.float32)], axis=1)


def _router(x, Wg, C):
    T = x.shape[0]
    outs = pl.pallas_call(
        functools.partial(_router_body, C),
        out_shape=[
            jax.ShapeDtypeStruct((T, _K), jnp.int32),
            jax.ShapeDtypeStruct((T, _K), jnp.float32),
            jax.ShapeDtypeStruct((T, _K), jnp.int32),
            jax.ShapeDtypeStruct((T, _K), jnp.int32),
            jax.ShapeDtypeStruct((T, _K), jnp.float32),
        ],
    )(x, Wg)
    return outs


def _dispatch(x, slot_f, keep_f, n_rows):
    T, D = x.shape
    n_pair = slot_f.shape[0]
    info = plsc.get_sparse_core_info()
    nw = info.num_cores * info.num_subcores
    tok_per_w = T // nw
    pairs_per_w = n_pair // nw
    mesh = plsc.VectorSubcoreMesh(core_axis_name="c", subcore_axis_name="s")

    @functools.partial(
        pl.kernel,
        mesh=mesh,
        out_type=jax.ShapeDtypeStruct((n_rows + 8, D), jnp.float32),
        scratch_types=[
            pltpu.VMEM((pairs_per_w,), jnp.int32),
            pltpu.VMEM((pairs_per_w,), jnp.int32),
            pltpu.VMEM((tok_per_w,), jnp.int32),
            pltpu.VMEM((tok_per_w,), jnp.int32),
            pltpu.VMEM((tok_per_w, D), jnp.float32),
            pltpu.SemaphoreType.DMA,
        ],
        compiler_params=pltpu.CompilerParams(needs_layout_passes=False),
    )
    def k(x_hbm, slot_hbm, keep_hbm, xe_hbm, slot_v, keep_v, idx0_v, idx1_v,
          rows_v, sem):
        wid = lax.axis_index("s") * info.num_cores + lax.axis_index("c")
        pb = pl.multiple_of(wid * pairs_per_w, pairs_per_w)
        tb = pl.multiple_of(wid * tok_per_w, tok_per_w)
        cp = pltpu.async_copy(x_hbm.at[pl.ds(tb, tok_per_w)], rows_v, sem)
        pltpu.sync_copy(slot_hbm.at[pl.ds(pb, pairs_per_w)], slot_v)
        pltpu.sync_copy(keep_hbm.at[pl.ds(pb, pairs_per_w)], keep_v)
        dump = jnp.full((16,), n_rows, jnp.int32)
        lanes = lax.broadcasted_iota(jnp.int32, (16,), 0)
        for j in range(tok_per_w // 16):
            two = jnp.full((16,), 2, jnp.int32)
            src_idx0 = jnp.full((16,), 32 * j, jnp.int32) + lanes * two
            src_idx1 = src_idx0 + 1
            s0 = plsc.load_gather(slot_v, [src_idx0])
            k0 = plsc.load_gather(keep_v, [src_idx0])
            s1 = plsc.load_gather(slot_v, [src_idx1])
            k1 = plsc.load_gather(keep_v, [src_idx1])
            idx0_v[pl.ds(j * 16, 16)] = jnp.where(k0 > 0, s0, dump)
            idx1_v[pl.ds(j * 16, 16)] = jnp.where(k1 > 0, s1, dump)
        cp.wait()
        pltpu.sync_copy(rows_v, xe_hbm.at[idx0_v])
        pltpu.sync_copy(rows_v, xe_hbm.at[idx1_v])

    return k(x, slot_f, keep_f)


def _ffn_body(xe_ref, w1_ref, b1_ref, w2_ref, b2_ref, out_ref):
    xb = xe_ref[...]
    h = jnp.dot(xb, w1_ref[0], preferred_element_type=jnp.float32)
    h = jnp.maximum(h + b1_ref[0], 0.0)
    out_ref[...] = (jnp.dot(h, w2_ref[0], preferred_element_type=jnp.float32)
                    + b2_ref[0])


def _ffn(xe, W1, b1, W2, b2, C):
    E, D, F = W1.shape
    out = pl.pallas_call(
        _ffn_body,
        grid=(E,),
        in_specs=[
            pl.BlockSpec((C, D), lambda e: (e, 0)),
            pl.BlockSpec((1, D, F), lambda e: (e, 0, 0)),
            pl.BlockSpec((1, 1, F), lambda e: (e, 0, 0)),
            pl.BlockSpec((1, F, D), lambda e: (e, 0, 0)),
            pl.BlockSpec((1, 1, D), lambda e: (e, 0, 0)),
        ],
        out_specs=pl.BlockSpec((C, D), lambda e: (e, 0)),
        out_shape=jax.ShapeDtypeStruct((E * C, D), jnp.float32),
        compiler_params=pltpu.CompilerParams(
            dimension_semantics=("arbitrary",)),
    )(xe, W1, b1.reshape(E, 1, F), W2, b2.reshape(E, 1, D))
    return out


def _combine(ye, slot_f, w_f, T):
    n_rows, D = ye.shape
    info = plsc.get_sparse_core_info()
    nw = info.num_cores * info.num_subcores
    tok_per_w = T // nw
    pairs_per_w = _K * tok_per_w
    tch = 16
    n_ch = tok_per_w // tch
    mesh = plsc.VectorSubcoreMesh(core_axis_name="c", subcore_axis_name="s")

    @functools.partial(
        pl.kernel,
        mesh=mesh,
        out_type=jax.ShapeDtypeStruct((T, D), jnp.float32),
        scratch_types=[
            pltpu.VMEM((pairs_per_w,), jnp.int32),
            pltpu.VMEM((pairs_per_w + 8,), jnp.float32),
            pltpu.VMEM((_K * tch,), jnp.int32),
            pltpu.VMEM((_K * tch,), jnp.int32),
            pltpu.VMEM((_K * tch, D), jnp.float32),
            pltpu.VMEM((_K * tch, D), jnp.float32),
            pltpu.VMEM((tch, D), jnp.float32),
            pltpu.VMEM((tch, D), jnp.float32),
            pltpu.SemaphoreType.DMA,
            pltpu.SemaphoreType.DMA,
            pltpu.SemaphoreType.DMA,
        ],
        compiler_params=pltpu.CompilerParams(needs_layout_passes=False),
    )
    def k(ye_hbm, slot_hbm, w_hbm, out_hbm, idx_v, w_v, idxc0_v, idxc1_v,
          rows0_v, rows1_v, outb0_v, outb1_v, semg0, semg1, semw):
        wid = lax.axis_index("s") * info.num_cores + lax.axis_index("c")
        pb = pl.multiple_of(wid * pairs_per_w, pairs_per_w)
        pltpu.sync_copy(slot_hbm.at[pl.ds(pb, pairs_per_w)], idx_v)
        pltpu.sync_copy(w_hbm.at[pl.ds(pb, pairs_per_w)],
                        w_v.at[pl.ds(8, pairs_per_w)])
        idxc = [idxc0_v, idxc1_v]
        rows = [rows0_v, rows1_v]
        outb = [outb0_v, outb1_v]
        semg = [semg0, semg1]

        def start_gather(ch):
            cur = ch % 2
            for j in range(_K * tch // 16):
                idxc[cur][pl.ds(j * 16, 16)] = idx_v[
                    pl.ds(ch * _K * tch + j * 16, 16)]
            return pltpu.async_copy(ye_hbm.at[idxc[cur]], rows[cur],
                                    semg[cur])

        gcp = start_gather(0)
        wcps = []
        for ch in range(n_ch):
            cur = ch % 2
            gcp.wait()
            if ch + 1 < n_ch:
                gcp = start_gather(ch + 1)
            wvecs = []
            for t in range(tch):
                w0 = plsc.load_gather(
                    w_v,
                    [jnp.full((16,), 8 + ch * _K * tch + 2 * t, jnp.int32)])
                w1 = plsc.load_gather(
                    w_v,
                    [jnp.full((16,), 8 + ch * _K * tch + 2 * t + 1,
                              jnp.int32)])
                wvecs.append((w0, w1))
            if ch >= 2:
                wcps[ch - 2].wait()

            def c_body(j, carry):
                o = pl.multiple_of(j * 16, 16)
                for t in range(tch):
                    w0, w1 = wvecs[t]
                    outb[cur][t, pl.ds(o, 16)] = (
                        rows[cur][2 * t, pl.ds(o, 16)] * w0
                        + rows[cur][2 * t + 1, pl.ds(o, 16)] * w1)
                return carry

            lax.fori_loop(0, D // 16, c_body, 0)
            tb = pl.multiple_of(wid * tok_per_w + ch * tch, tch)
            wcps.append(
                pltpu.async_copy(outb[cur], out_hbm.at[pl.ds(tb, tch)],
                                 semw))
        wcps[n_ch - 2].wait()
        wcps[n_ch - 1].wait()

    return k(ye, slot_f, w_f)


def kernel(x, Wg, W1, b1, W2, b2):
    T, D = x.shape
    E = Wg.shape[1]
    C = int(math.ceil(T * _K / E * _CAP_FACTOR))
    eidx, gates, maski, slot, wcomb = _router(x, Wg, C)
    slot_f = slot.reshape(-1)
    keep_f = maski.reshape(-1)
    xe = _dispatch(x, slot_f, keep_f, E * C)
    ye = _ffn(xe, W1, b1, W2, b2, C)
    out = _combine(ye, slot_f, wcomb.reshape(-1), T)
    return out, maski.astype(bool), eidx, gates
